# baseline (device time: 189340 ns/iter reference)
import jax
import jax.numpy as jnp
from jax import lax
from jax.experimental import pallas as pl
from jax.experimental.pallas import tpu as pltpu

N_DEV = 16
HOPS = 8
N_SUB = 4


def kernel(x, w_mat):
    x = x.astype(jnp.bfloat16)
    w_mat = w_mat.astype(jnp.bfloat16)
    m_per, k = x.shape
    _, n_per = w_mat.shape
    mq = m_per // N_SUB

    R_LONG = (0, 1)
    L_LONG = (2, 3)

    def body(x_ref, w_ref, out_ref, comm_ref, ss_r, rs_r, ss_l, rs_l):
        my_pos = lax.axis_index("i")
        left = lax.rem(my_pos - 1 + N_DEV, N_DEV)
        right = lax.rem(my_pos + 1, N_DEV)

        barrier_sem = pltpu.get_barrier_semaphore()
        for nbr in [left, right]:
            pl.semaphore_signal(
                barrier_sem, inc=1,
                device_id=(nbr,), device_id_type=pl.DeviceIdType.MESH,
            )
        comm_ref[0] = x_ref[...]
        pl.semaphore_wait(barrier_sem, 2)

        def mk(src_slot, dst_slot, sub, ss, rs, hop, target):
            rows = pl.ds(sub * mq, mq)
            return pltpu.make_async_remote_copy(
                src_ref=comm_ref.at[src_slot, rows],
                dst_ref=comm_ref.at[dst_slot, rows],
                send_sem=ss.at[sub, hop - 1],
                recv_sem=rs.at[sub, hop - 1],
                device_id=(target,),
                device_id_type=pl.DeviceIdType.MESH,
            )

        def gemm(slot, origin):
            out_ref[pl.ds(origin * m_per, m_per), :] = jnp.dot(
                comm_ref[slot], w_ref[...],
                preferred_element_type=jnp.float32,
            )

        rd_r = {}
        rd_l = {}
        for h in range(1, HOPS + 1):
            for s in range(N_SUB):
                if h > 1:
                    rd_r[(s, h - 1)].wait_recv()
                if h <= 7 or s in R_LONG:
                    rd_r[(s, h)] = mk(h - 1, h, s, ss_r, rs_r, h, right)
                    rd_r[(s, h)].start()

            for s in range(N_SUB):
                if h > 1:
                    rd_l[(s, h - 1)].wait_recv()
                if h <= 7 or s in L_LONG:
                    src = 0 if h == 1 else HOPS + h - 1
                    dst = HOPS + h
                    if h == 8:
                        src, dst = HOPS + 7, HOPS
                    rd_l[(s, h)] = mk(src, dst, s, ss_l, rs_l, h, left)
                    rd_l[(s, h)].start()

            if h == 1:
                gemm(0, my_pos)
            else:
                gemm(h - 1, lax.rem(my_pos - (h - 1) + N_DEV, N_DEV))
                gemm(HOPS + h - 1, lax.rem(my_pos + (h - 1), N_DEV))

        for s in R_LONG:
            rd_r[(s, HOPS)].wait_recv()
        for s in L_LONG:
            rd_l[(s, HOPS)].wait_recv()
        gemm(HOPS, lax.rem(my_pos - HOPS + N_DEV, N_DEV))

        for rd in list(rd_r.values()) + list(rd_l.values()):
            rd.wait_send()

    return pl.pallas_call(
        body,
        out_shape=jax.ShapeDtypeStruct((N_DEV * m_per, n_per), jnp.float32),
        in_specs=[
            pl.BlockSpec(memory_space=pltpu.VMEM),
            pl.BlockSpec(memory_space=pltpu.VMEM),
        ],
        out_specs=pl.BlockSpec(memory_space=pltpu.VMEM),
        scratch_shapes=[
            pltpu.VMEM((N_DEV, m_per, k), jnp.bfloat16),
            pltpu.SemaphoreType.DMA((N_SUB, HOPS)),
            pltpu.SemaphoreType.DMA((N_SUB, HOPS)),
            pltpu.SemaphoreType.DMA((N_SUB, HOPS)),
            pltpu.SemaphoreType.DMA((N_SUB, HOPS)),
        ],
        compiler_params=pltpu.CompilerParams(collective_id=0),
    )(x, w_mat)


# device time: 178545 ns/iter; 1.0605x vs baseline; 1.0605x over previous
import jax
import jax.numpy as jnp
from jax import lax
from jax.experimental import pallas as pl
from jax.experimental.pallas import tpu as pltpu

N_DEV = 16


def kernel(x, w_mat):
    x = x.astype(jnp.bfloat16)
    w_mat = w_mat.astype(jnp.bfloat16)
    m_per, k = x.shape
    _, n_per = w_mat.shape

    def body(x_ref, w_ref, out_ref, comm_ref,
             ss_fo, rs_fo, ss_fp, rs_fp, ss_bo, rs_bo, ss_bp, rs_bp,
             ss_x, rs_x):
        my_pos = lax.axis_index("i")
        z = my_pos // 4
        i = lax.rem(my_pos, 4)
        xc = jnp.where((i == 1) | (i == 2), 1, 0)
        yc = jnp.where(i >= 2, 1, 0)
        i0 = xc
        i1 = 3 - xc
        j = jnp.where(yc == 0, z, 7 - z)

        def ring_pos(jj):
            jj = lax.rem(jj + 16, 8)
            return jnp.where(jj < 4, 4 * jj + i0, 4 * (7 - jj) + i1)

        fwd = ring_pos(j + 1)
        bwd = ring_pos(j - 1)
        partner = 4 * z + jnp.bitwise_xor(i, 1)

        barrier_sem = pltpu.get_barrier_semaphore()
        for nbr in [fwd, bwd, partner]:
            pl.semaphore_signal(
                barrier_sem, inc=1,
                device_id=(nbr,), device_id_type=pl.DeviceIdType.MESH,
            )
        comm_ref[0] = x_ref[...]
        pl.semaphore_wait(barrier_sem, 3)

        sends = []

        def start(src, dst, ss, rs, idx, target):
            rd = pltpu.make_async_remote_copy(
                src_ref=comm_ref.at[src],
                dst_ref=comm_ref.at[dst],
                send_sem=ss.at[idx],
                recv_sem=rs.at[idx],
                device_id=(target,),
                device_id_type=pl.DeviceIdType.MESH,
            )
            rd.start()
            sends.append(rd)
            return rd

        def gemm(slot, origin):
            out_ref[pl.ds(origin * m_per, m_per), :] = jnp.dot(
                comm_ref[slot], w_ref[...],
                preferred_element_type=jnp.float32,
            )

        x_rd = start(0, 1, ss_x, rs_x, 0, partner)
        fo = {1: start(0, 2, ss_fo, rs_fo, 0, fwd)}
        bo = {1: start(0, 9, ss_bo, rs_bo, 0, bwd)}
        gemm(0, my_pos)

        x_rd.wait_recv()
        fp = {1: start(1, 3, ss_fp, rs_fp, 0, fwd)}
        bp = {1: start(1, 10, ss_bp, rs_bp, 0, bwd)}
        gemm(1, partner)

        for h in range(2, 5):
            fo[h - 1].wait_recv()
            fo[h] = start(2 * (h - 1), 2 * h, ss_fo, rs_fo, h - 1, fwd)
            bo[h - 1].wait_recv()
            if h <= 3:
                bo[h] = start(2 * (h - 1) + 7, 2 * h + 7,
                              ss_bo, rs_bo, h - 1, bwd)
            fp[h - 1].wait_recv()
            if h <= 3:
                fp[h] = start(2 * h - 1, 2 * h + 1,
                              ss_fp, rs_fp, h - 1, fwd)
            bp[h - 1].wait_recv()
            bp[h] = start(2 * (h - 1) + 8, 15 if h == 4 else 2 * h + 8,
                          ss_bp, rs_bp, h - 1, bwd)

            qf = ring_pos(j - (h - 1))
            qb = ring_pos(j + (h - 1))
            gemm(2 * (h - 1), qf)
            gemm(2 * (h - 1) + 1, jnp.bitwise_xor(qf, 1))
            gemm(2 * (h - 1) + 7, qb)
            gemm(2 * (h - 1) + 8, jnp.bitwise_xor(qb, 1))

        fo[4].wait_recv()
        bp[4].wait_recv()
        q4 = ring_pos(j + 4)
        gemm(8, q4)
        gemm(15, jnp.bitwise_xor(q4, 1))

        for rd in sends:
            rd.wait_send()

    return pl.pallas_call(
        body,
        out_shape=jax.ShapeDtypeStruct((N_DEV * m_per, n_per), jnp.float32),
        in_specs=[
            pl.BlockSpec(memory_space=pltpu.VMEM),
            pl.BlockSpec(memory_space=pltpu.VMEM),
        ],
        out_specs=pl.BlockSpec(memory_space=pltpu.VMEM),
        scratch_shapes=[
            pltpu.VMEM((N_DEV, m_per, k), jnp.bfloat16),
            pltpu.SemaphoreType.DMA((4,)),
            pltpu.SemaphoreType.DMA((4,)),
            pltpu.SemaphoreType.DMA((3,)),
            pltpu.SemaphoreType.DMA((3,)),
            pltpu.SemaphoreType.DMA((3,)),
            pltpu.SemaphoreType.DMA((3,)),
            pltpu.SemaphoreType.DMA((4,)),
            pltpu.SemaphoreType.DMA((4,)),
            pltpu.SemaphoreType.DMA((1,)),
            pltpu.SemaphoreType.DMA((1,)),
        ],
        compiler_params=pltpu.CompilerParams(collective_id=0),
    )(x, w_mat)
